# baseline (speedup 1.0000x reference)
import jax
import jax.numpy as jnp
from jax import lax
from jax.experimental import pallas as pl
from jax.experimental.pallas import tpu as pltpu
from jax.experimental.pallas import tpu_sc as plsc

B = 16384
K = 32
NC = 2
NS = 16
NW = NC * NS
ROWS = B // NW
GROUP = 16
NGROUPS = ROWS // GROUP
CHUNK = 128
NCHUNK = ROWS // CHUNK


def _fm_kernel(xu_hbm, xm_hbm, xg_hbm, uemb_hbm, memb_hbm, gemb_hbm,
               ubias_hbm, mbias_hbm, lin_hbm, out_hbm,
               idx_u, idx_m, idx_g, uv, mv, gv, ubv, mbv, outv, linv, sem):
    wid = lax.axis_index("s") * NC + lax.axis_index("c")
    base = wid * ROWS

    for j in range(NCHUNK):
        pltpu.sync_copy(xu_hbm.at[pl.ds(base + j * CHUNK, CHUNK)], idx_u.at[j])
        pltpu.sync_copy(xm_hbm.at[pl.ds(base + j * CHUNK, CHUNK)], idx_m.at[j])
        pltpu.sync_copy(xg_hbm.at[pl.ds(base + j * CHUNK, CHUNK)], idx_g.at[j])
    pltpu.sync_copy(lin_hbm, linv)

    cps = []
    for j in range(NCHUNK):
        s = pl.ds(j * CHUNK, CHUNK)
        cps.append(pltpu.async_copy(uemb_hbm.at[idx_u.at[j]], uv.at[s, :], sem))
        cps.append(pltpu.async_copy(memb_hbm.at[idx_m.at[j]], mv.at[s, :], sem))
        cps.append(pltpu.async_copy(gemb_hbm.at[idx_g.at[j]], gv.at[s, :], sem))
        cps.append(pltpu.async_copy(ubias_hbm.at[idx_u.at[j]], ubv.at[s], sem))
        cps.append(pltpu.async_copy(mbias_hbm.at[idx_m.at[j]], mbv.at[s], sem))
    for cp in cps:
        cp.wait()

    lv = linv[pl.ds(0, 16)]
    w0 = lv[0]
    w1 = lv[1]
    w2 = lv[2]
    lb = lv[3]
    lane = lax.iota(jnp.int32, 16)

    @pl.loop(0, NGROUPS)
    def _(g):
        rbase = g * GROUP
        acc = jnp.zeros((16,), jnp.float32)
        for j in range(GROUP):
            r = rbase + j
            u = uv[r, :]
            m = mv[r, :]
            gg = gv[r, :]
            t = u * (m + gg) + m * gg
            ta, tb = plsc.unpack(t, format=plsc.PackFormat.INTERLEAVED)
            ts = jnp.sum(ta + tb, axis=0)
            acc = jnp.where(lane == j, ts, acc)
        c = g // (CHUNK // GROUP)
        off = (g % (CHUNK // GROUP)) * GROUP
        iu = idx_u[c, pl.ds(off, 16)].astype(jnp.float32)
        im = idx_m[c, pl.ds(off, 16)].astype(jnp.float32)
        ig = idx_g[c, pl.ds(off, 16)].astype(jnp.float32)
        lin = iu * w0 + im * w1 + ig * w2 + lb
        pre = acc + ubv[pl.ds(rbase, 16)] + mbv[pl.ds(rbase, 16)] + lin
        pre = jnp.minimum(jnp.maximum(pre, -30.0), 30.0)
        outv[pl.ds(rbase, 16)] = 1.0 / (1.0 + jnp.exp(-pre))

    pltpu.sync_copy(outv, out_hbm.at[pl.ds(base, ROWS)])


def kernel(x, user_emb, movie_emb, genre_emb, user_bias, movie_bias, lin_w, lin_b):
    xu = x[:, 0]
    xm = x[:, 1]
    xg = x[:, 2]
    ue = user_emb.astype(jnp.bfloat16)
    me = movie_emb.astype(jnp.bfloat16)
    ge = genre_emb.astype(jnp.bfloat16)
    ub = user_bias.reshape(-1)
    mb = movie_bias.reshape(-1)
    lin = jnp.concatenate(
        [lin_w.reshape(-1), lin_b.reshape(-1), jnp.zeros((12,), jnp.float32)]
    )

    cp = pltpu.CompilerParams(
        needs_layout_passes=False, use_tc_tiling_on_sc=False
    )
    mesh = plsc.VectorSubcoreMesh(core_axis_name="c", subcore_axis_name="s")
    fm = pl.kernel(
        _fm_kernel,
        out_type=jax.ShapeDtypeStruct((B,), jnp.float32),
        mesh=mesh,
        scratch_types=[
            pltpu.VMEM((NCHUNK, CHUNK), jnp.int32),
            pltpu.VMEM((NCHUNK, CHUNK), jnp.int32),
            pltpu.VMEM((NCHUNK, CHUNK), jnp.int32),
            pltpu.VMEM((ROWS, K), jnp.bfloat16),
            pltpu.VMEM((ROWS, K), jnp.bfloat16),
            pltpu.VMEM((ROWS, K), jnp.bfloat16),
            pltpu.VMEM((ROWS,), jnp.float32),
            pltpu.VMEM((ROWS,), jnp.float32),
            pltpu.VMEM((ROWS,), jnp.float32),
            pltpu.VMEM((16,), jnp.float32),
            pltpu.SemaphoreType.DMA,
        ],
        compiler_params=cp,
    )
    out = fm(xu, xm, xg, ue, me, ge, ub, mb, lin)
    return out.reshape(B, 1)


# speedup vs baseline: 1.7089x; 1.7089x over previous
import jax
import jax.numpy as jnp
from jax import lax
from jax.experimental import pallas as pl
from jax.experimental.layout import Format, Layout, with_layout_constraint
from jax.experimental.pallas import tpu as pltpu
from jax.experimental.pallas import tpu_sc as plsc

B = 16384
K = 32
NC = 2
NS = 16
NW = NC * NS
ROWS = B // NW
GROUP = 16
NGROUPS = ROWS // GROUP
CHUNK = 128
NCHUNK = ROWS // CHUNK


def _fm_kernel(xu_hbm, xm_hbm, xg_hbm, uemb_hbm, memb_hbm, gemb_hbm,
               ubias_hbm, mbias_hbm, lin_hbm, out_hbm,
               idx_u, idx_m, idx_g, uv, mv, gv, ubv, mbv, outv, linv, sem):
    wid = lax.axis_index("s") * NC + lax.axis_index("c")
    base = wid * ROWS

    for j in range(NCHUNK):
        pltpu.sync_copy(xu_hbm.at[pl.ds(base + j * CHUNK, CHUNK)], idx_u.at[j])
        pltpu.sync_copy(xm_hbm.at[pl.ds(base + j * CHUNK, CHUNK)], idx_m.at[j])
        pltpu.sync_copy(xg_hbm.at[pl.ds(base + j * CHUNK, CHUNK)], idx_g.at[j])
    pltpu.sync_copy(lin_hbm, linv)

    cps = []
    for j in range(NCHUNK):
        s = pl.ds(j * CHUNK, CHUNK)
        cps.append(pltpu.async_copy(uemb_hbm.at[idx_u.at[j]], uv.at[s, :], sem))
        cps.append(pltpu.async_copy(memb_hbm.at[idx_m.at[j]], mv.at[s, :], sem))
        cps.append(pltpu.async_copy(gemb_hbm.at[idx_g.at[j]], gv.at[s, :], sem))
        cps.append(pltpu.async_copy(ubias_hbm.at[idx_u.at[j]], ubv.at[s], sem))
        cps.append(pltpu.async_copy(mbias_hbm.at[idx_m.at[j]], mbv.at[s], sem))
    for cp in cps:
        cp.wait()

    lv = linv[pl.ds(0, 16)]
    w0 = lv[0]
    w1 = lv[1]
    w2 = lv[2]
    lb = lv[3]
    lane = lax.iota(jnp.int32, 16)

    @pl.loop(0, NGROUPS)
    def _(g):
        rbase = g * GROUP
        acc = jnp.zeros((16,), jnp.float32)
        for j in range(GROUP):
            r = rbase + j
            u = uv[r, :]
            m = mv[r, :]
            gg = gv[r, :]
            t = u * (m + gg) + m * gg
            ta, tb = plsc.unpack(t, format=plsc.PackFormat.INTERLEAVED)
            ts = jnp.sum(ta + tb, axis=0)
            acc = jnp.where(lane == j, ts, acc)
        c = g // (CHUNK // GROUP)
        off = (g % (CHUNK // GROUP)) * GROUP
        iu = idx_u[c, pl.ds(off, 16)].astype(jnp.float32)
        im = idx_m[c, pl.ds(off, 16)].astype(jnp.float32)
        ig = idx_g[c, pl.ds(off, 16)].astype(jnp.float32)
        lin = iu * w0 + im * w1 + ig * w2 + lb
        pre = acc + ubv[pl.ds(rbase, 16)] + mbv[pl.ds(rbase, 16)] + lin
        pre = jnp.minimum(jnp.maximum(pre, -30.0), 30.0)
        outv[pl.ds(rbase, 16)] = 1.0 / (1.0 + jnp.exp(-pre))

    pltpu.sync_copy(outv, out_hbm.at[pl.ds(base, ROWS)])


def kernel(x, user_emb, movie_emb, genre_emb, user_bias, movie_bias, lin_w, lin_b):
    xu = x[:, 0]
    xm = x[:, 1]
    xg = x[:, 2]
    rowmajor = Layout((0, 1))

    def to_bf16_rowmajor(t):
        return with_layout_constraint(t.astype(jnp.bfloat16), rowmajor)

    ue = to_bf16_rowmajor(user_emb)
    me = to_bf16_rowmajor(movie_emb)
    ge = to_bf16_rowmajor(genre_emb)
    ub = user_bias.reshape(-1)
    mb = movie_bias.reshape(-1)
    lin = jnp.concatenate(
        [lin_w.reshape(-1), lin_b.reshape(-1), jnp.zeros((12,), jnp.float32)]
    )

    cp = pltpu.CompilerParams(
        needs_layout_passes=False, use_tc_tiling_on_sc=False
    )
    mesh = plsc.VectorSubcoreMesh(core_axis_name="c", subcore_axis_name="s")
    fm = pl.kernel(
        _fm_kernel,
        out_type=jax.ShapeDtypeStruct((B,), jnp.float32),
        mesh=mesh,
        scratch_types=[
            pltpu.VMEM((NCHUNK, CHUNK), jnp.int32),
            pltpu.VMEM((NCHUNK, CHUNK), jnp.int32),
            pltpu.VMEM((NCHUNK, CHUNK), jnp.int32),
            pltpu.VMEM((ROWS, K), jnp.bfloat16),
            pltpu.VMEM((ROWS, K), jnp.bfloat16),
            pltpu.VMEM((ROWS, K), jnp.bfloat16),
            pltpu.VMEM((ROWS,), jnp.float32),
            pltpu.VMEM((ROWS,), jnp.float32),
            pltpu.VMEM((ROWS,), jnp.float32),
            pltpu.VMEM((16,), jnp.float32),
            pltpu.SemaphoreType.DMA,
        ],
        compiler_params=cp,
    )
    out = fm(xu, xm, xg, ue, me, ge, ub, mb, lin)
    return out.reshape(B, 1)
